# baseline (speedup 1.0000x reference)
import functools

import jax
import jax.numpy as jnp
from jax import lax
from jax.experimental import pallas as pl
from jax.experimental.pallas import tpu as pltpu
from jax.experimental.pallas import tpu_sc as plsc

_N = 20000
_G = 50
_ROWS = 160
_LANES = 128
_NPAD = _ROWS * _LANES

_POS_UPPER = 128
_NEG_THR = 0.3
_POS_THR = 0.5

_W = 16
_CHUNK = _NPAD // _W
_VPW = _CHUNK // 16


def _tc_stage(a_ref, s_ref, r_ref, gts_ref, arr_ref, gx_ref, iou_s, cm_s,
              pm_s):
    f32 = jnp.float32
    a1 = a_ref[0]
    a2 = a_ref[1]
    a3 = a_ref[2]
    a4 = a_ref[3]
    area_a = (a3 - a1) * (a4 - a2)

    row_i = lax.broadcasted_iota(jnp.int32, (_ROWS, _LANES), 0)
    col_i = lax.broadcasted_iota(jnp.int32, (_ROWS, _LANES), 1)
    idx = row_i * _LANES + col_i
    idxf = idx.astype(f32)
    valid = idx < _N

    neg_one = jnp.full((_ROWS, _LANES), -1.0, f32)
    zeros = jnp.zeros((_ROWS, _LANES), f32)

    def body1(g, carry):
        run_max, mx1, my1, mx2, my2 = carry
        gx1 = gts_ref[g, 0]
        gy1 = gts_ref[g, 1]
        gx2 = gts_ref[g, 2]
        gy2 = gts_ref[g, 3]
        garea = (gx2 - gx1) * (gy2 - gy1)
        w = jnp.maximum(jnp.minimum(a3, gx2) - jnp.maximum(a1, gx1), 0.0)
        h = jnp.maximum(jnp.minimum(a4, gy2) - jnp.maximum(a2, gy1), 0.0)
        inter = w * h
        union = area_a + garea - inter
        iou = jnp.where(valid, inter / union, -1.0)
        iou_s[g] = iou
        cm_s[g] = jnp.max(iou, axis=0, keepdims=True)
        better = iou > run_max
        run_max = jnp.where(better, iou, run_max)
        mx1 = jnp.where(better, gx1, mx1)
        my1 = jnp.where(better, gy1, my1)
        mx2 = jnp.where(better, gx2, mx2)
        my2 = jnp.where(better, gy2, my2)
        return run_max, mx1, my1, mx2, my2

    init = (neg_one, zeros, zeros, zeros + 1.0, zeros + 1.0)
    run_max, mx1, my1, mx2, my2 = lax.fori_loop(0, _G, body1, init)

    cm = cm_s[...]
    colmax = jnp.max(cm, axis=2, keepdims=True)
    cm_s[...] = jnp.broadcast_to(colmax, (_G, 1, _LANES))

    def body2(g, carry):
        iou_g = iou_s[g]
        cmax_b = jnp.broadcast_to(cm_s[g], (_ROWS, _LANES))
        cand = jnp.where(iou_g == cmax_b, idxf, 3e7)
        pm_s[g] = jnp.min(cand, axis=0, keepdims=True)
        return carry

    lax.fori_loop(0, _G, body2, 0)
    pmv = pm_s[...]
    gx_ref[...] = jnp.broadcast_to(
        jnp.min(pmv, axis=2, keepdims=True), (_G, 1, _LANES))

    pos_cand = jnp.logical_and(valid, run_max > _POS_THR).astype(f32)
    neg_cand = jnp.logical_and(valid, run_max < _NEG_THR).astype(f32)

    s0 = s_ref[0]
    s1 = s_ref[1]
    sm = jnp.maximum(s0, s1)
    lse = sm + jnp.log(jnp.exp(s0 - sm) + jnp.exp(s1 - sm))
    logp0_score = s0 - lse
    logp1_score = s1 - lse
    am = jnp.maximum(jnp.maximum(a1, a2), jnp.maximum(a3, a4))
    alse = am + jnp.log(jnp.exp(a1 - am) + jnp.exp(a2 - am)
                        + jnp.exp(a3 - am) + jnp.exp(a4 - am))
    logp0_anch = a1 - alse

    aw = a3 - a1
    ah = a4 - a2
    acx = a1 + aw * 0.5
    acy = a2 + ah * 0.5
    gw = mx2 - mx1
    gh = my2 - my1
    gcx = mx1 + gw * 0.5
    gcy = my1 + gh * 0.5
    tx = (gcx - acx) / aw
    ty = (gcy - acy) / ah
    tw = jnp.log(gw / aw)
    th = jnp.log(gh / ah)

    def sl1(pred, tgt):
        d = pred - tgt
        ad = jnp.abs(d)
        return jnp.where(ad < 1.0, 0.5 * d * d, ad - 0.5)

    reg_sum = (sl1(r_ref[0], tx) + sl1(r_ref[1], ty)
               + sl1(r_ref[2], tw) + sl1(r_ref[3], th))

    arr_ref[0] = pos_cand
    arr_ref[1] = neg_cand
    arr_ref[2] = logp0_score
    arr_ref[3] = logp0_anch
    arr_ref[4] = reg_sum
    arr_ref[5] = logp1_score


def _sc_stage(pos_hbm, neg_hbm, ts_hbm, ta_hbm, rg_hbm, tn_hbm, gti_hbm,
              out_hbm, posr, negr, tsr, tar, rgr, tnr,
              gti, stg, outv, ov2, shared):
    f32 = jnp.float32
    w = lax.axis_index("s")
    base = w * _CHUNK
    srcs = (pos_hbm, neg_hbm, ts_hbm, ta_hbm, rg_hbm, tn_hbm)
    bufs = (posr, negr, tsr, tar, rgr, tnr)
    for j in range(6):
        pltpu.sync_copy(srcs[j].at[pl.ds(base, _CHUNK)], bufs[j])
    pltpu.sync_copy(gti_hbm, gti)

    lane = lax.iota(jnp.int32, 16)
    ones16 = jnp.ones((16,), f32)
    zeros16 = jnp.zeros((16,), f32)

    for v in range(4):
        gv = gti[pl.ds(v * 16, 16)]
        lv = gv - base
        m = jnp.logical_and(lv >= 0, lv < _CHUNK)
        lv = jnp.where(m, lv, 0)
        plsc.store_scatter(posr, [lv], ones16, mask=m)
        plsc.store_scatter(negr, [lv], zeros16, mask=m)

    def cbody(v, carry):
        accp, accq = carry
        accp = accp + posr[pl.ds(v * 16, 16)]
        accq = accq + negr[pl.ds(v * 16, 16)]
        return accp, accq

    accp, accq = lax.fori_loop(0, _VPW, cbody, (zeros16, zeros16))
    cp = jnp.sum(accp)
    cq = jnp.sum(accq)

    rowv = jnp.where(lane == 0, cp, jnp.where(lane == 1, cq, 0.0))
    ov2[pl.ds(0, 16)] = rowv
    pltpu.sync_copy(ov2, shared.at[w])
    plsc.subcore_barrier()
    pltpu.sync_copy(shared, stg)

    acc = zeros16
    myoff = zeros16
    for wp in range(_W):
        myoff = jnp.where(w == wp, acc, myoff)
        acc = acc + stg[wp, pl.ds(0, 16)]
    p_full = jnp.sum(jnp.where(lane == 0, acc, 0.0))
    q_full = jnp.sum(jnp.where(lane == 1, acc, 0.0))
    offp = jnp.sum(jnp.where(lane == 0, myoff, 0.0))
    offq = jnp.sum(jnp.where(lane == 1, myoff, 0.0))

    k_pos = jnp.minimum(p_full, float(_POS_UPPER))
    score_bug = p_full >= float(_POS_UPPER)
    neg_bound = jnp.where(p_full < float(_POS_UPPER), p_full,
                          float(_POS_UPPER))
    q_eff = jnp.minimum(neg_bound, q_full)

    def sbody(v, carry):
        rp, rq, asp, asr, asn = carry
        pv = posr[pl.ds(v * 16, 16)]
        nv = negr[pl.ds(v * 16, 16)]
        ts = tsr[pl.ds(v * 16, 16)]
        ta = tar[pl.ds(v * 16, 16)]
        rg = rgr[pl.ds(v * 16, 16)]
        tn = tnr[pl.ds(v * 16, 16)]
        rankp = plsc.cumsum(pv) + rp
        selp = jnp.logical_and(pv > 0.0, rankp <= k_pos)
        term = jnp.where(score_bug, ta, ts)
        asp = asp + jnp.where(selp, term, 0.0)
        asr = asr + jnp.where(selp, rg, 0.0)
        rankq = plsc.cumsum(nv) + rq
        selq = jnp.logical_and(nv > 0.0, rankq <= q_eff)
        asn = asn + jnp.where(selq, tn, 0.0)
        return (rp + jnp.sum(pv), rq + jnp.sum(nv), asp, asr, asn)

    _, _, asp, asr, asn = lax.fori_loop(
        0, _VPW, sbody, (offp, offq, zeros16, zeros16, zeros16))
    sp = jnp.sum(asp)
    sr = jnp.sum(asr)
    sn = jnp.sum(asn)

    rowv = jnp.where(lane == 0, sp,
                     jnp.where(lane == 1, sr,
                               jnp.where(lane == 2, sn, 0.0)))
    ov2[pl.ds(0, 16)] = rowv
    pltpu.sync_copy(ov2, shared.at[w])
    plsc.subcore_barrier()

    @pl.when(w == 0)
    def _():
        pltpu.sync_copy(shared, stg)
        tot = jnp.zeros((16,), f32)
        for wp in range(_W):
            tot = tot + stg[wp, pl.ds(0, 16)]
        sp_t = jnp.sum(jnp.where(lane == 0, tot, 0.0))
        sr_t = jnp.sum(jnp.where(lane == 1, tot, 0.0))
        sn_t = jnp.sum(jnp.where(lane == 2, tot, 0.0))
        ones = jnp.ones((16,), f32)
        loss_v = ((ones * sr_t - ones * sp_t) / (ones * k_pos)
                  - (ones * sn_t) / (ones * q_eff))
        outv[...] = loss_v
        pltpu.sync_copy(outv, out_hbm)


@jax.jit
def kernel(score_pred, reg_pred, anchors, gts):
    pad = _NPAD - _N
    a_t = jnp.pad(anchors.T, ((0, 0), (0, pad)))
    a_t = a_t.at[2:, _N:].set(1.0)
    s_t = jnp.pad(score_pred.T, ((0, 0), (0, pad)))
    r_t = jnp.pad(reg_pred.T, ((0, 0), (0, pad)))
    a3 = a_t.reshape(4, _ROWS, _LANES)
    s3 = s_t.reshape(2, _ROWS, _LANES)
    r3 = r_t.reshape(4, _ROWS, _LANES)

    arr, gx = pl.pallas_call(
        _tc_stage,
        out_shape=[
            jax.ShapeDtypeStruct((6, _ROWS, _LANES), jnp.float32),
            jax.ShapeDtypeStruct((_G, 1, _LANES), jnp.float32),
        ],
        in_specs=[
            pl.BlockSpec(memory_space=pltpu.VMEM),
            pl.BlockSpec(memory_space=pltpu.VMEM),
            pl.BlockSpec(memory_space=pltpu.VMEM),
            pl.BlockSpec(memory_space=pltpu.SMEM),
        ],
        out_specs=[
            pl.BlockSpec(memory_space=pltpu.VMEM),
            pl.BlockSpec(memory_space=pltpu.VMEM),
        ],
        scratch_shapes=[
            pltpu.VMEM((_G, _ROWS, _LANES), jnp.float32),
            pltpu.VMEM((_G, 1, _LANES), jnp.float32),
            pltpu.VMEM((_G, 1, _LANES), jnp.float32),
        ],
    )(a3, s3, r3, gts)

    arr2 = arr.reshape(6, _NPAD)
    gidx = gx.reshape(_G, _LANES)[:, 0].astype(jnp.int32)
    gti64 = jnp.full((64,), -1, jnp.int32).at[:_G].set(gidx)

    mesh = plsc.VectorSubcoreMesh(
        core_axis_name="c", subcore_axis_name="s", num_cores=1,
        num_subcores=_W)
    out = pl.kernel(
        _sc_stage,
        out_type=jax.ShapeDtypeStruct((16,), jnp.float32),
        mesh=mesh,
        compiler_params=pltpu.CompilerParams(needs_layout_passes=False),
        scratch_types=[
            pltpu.VMEM((_CHUNK,), jnp.float32),
            pltpu.VMEM((_CHUNK,), jnp.float32),
            pltpu.VMEM((_CHUNK,), jnp.float32),
            pltpu.VMEM((_CHUNK,), jnp.float32),
            pltpu.VMEM((_CHUNK,), jnp.float32),
            pltpu.VMEM((_CHUNK,), jnp.float32),
            pltpu.VMEM((64,), jnp.int32),
            pltpu.VMEM((_W, 128), jnp.float32),
            pltpu.VMEM((16,), jnp.float32),
            pltpu.VMEM((128,), jnp.float32),
            pltpu.VMEM_SHARED((_W, 128), jnp.float32),
        ],
    )(arr2[0], arr2[1], arr2[2], arr2[3], arr2[4], arr2[5], gti64)
    return out[0]
